# initial kernel scaffold (unmeasured)
import jax
import jax.numpy as jnp
from jax import lax
from jax.experimental import pallas as pl
from jax.experimental.pallas import tpu as pltpu

N_DEV = 4


def _ring_allgather(x):
    m_per, k = x.shape
    m_total = N_DEV * m_per

    def body(x_ref, xg_ref, copy_sem, send_sems, recv_sems):
        my = lax.axis_index("i")
        left = (my - 1) % N_DEV
        right = (my + 1) % N_DEV

        barrier_sem = pltpu.get_barrier_semaphore()
        pl.semaphore_signal(
            barrier_sem, inc=1, device_id=(left,),
            device_id_type=pl.DeviceIdType.MESH,
        )
        pl.semaphore_signal(
            barrier_sem, inc=1, device_id=(right,),
            device_id_type=pl.DeviceIdType.MESH,
        )
        pl.semaphore_wait(barrier_sem, 2)

        cp = pltpu.make_async_copy(
            x_ref, xg_ref.at[pl.ds(my * m_per, m_per), :], copy_sem
        )
        cp.start()
        cp.wait()

        for h in range(N_DEV - 1):
            origin = (my - h) % N_DEV
            rows = pl.ds(origin * m_per, m_per)
            rdma = pltpu.make_async_remote_copy(
                src_ref=xg_ref.at[rows, :],
                dst_ref=xg_ref.at[rows, :],
                send_sem=send_sems.at[h],
                recv_sem=recv_sems.at[h],
                device_id=(right,),
                device_id_type=pl.DeviceIdType.MESH,
            )
            rdma.start()
            rdma.wait()

    return pl.pallas_call(
        body,
        out_shape=jax.ShapeDtypeStruct((m_total, k), x.dtype),
        in_specs=[pl.BlockSpec(memory_space=pltpu.ANY)],
        out_specs=pl.BlockSpec(memory_space=pltpu.ANY),
        scratch_shapes=[
            pltpu.SemaphoreType.DMA,
            pltpu.SemaphoreType.DMA((N_DEV - 1,)),
            pltpu.SemaphoreType.DMA((N_DEV - 1,)),
        ],
        compiler_params=pltpu.CompilerParams(collective_id=0),
    )(x)


def kernel(x, w_mat):
    x_full = _ring_allgather(x)
    y = x_full @ w_mat
    return y * jax.nn.sigmoid(y)


# baseline (device time: 668985 ns/iter reference)
import jax
import jax.numpy as jnp
from jax import lax
from jax.experimental import pallas as pl
from jax.experimental.pallas import tpu as pltpu

N_DEV = 4


def _ring_allgather(x):
    m_per, k = x.shape
    m_total = N_DEV * m_per

    def body(x_ref, xg_ref, copy_sem, send_sems, recv_sems):
        my = lax.axis_index("i")
        left = (my - 1) % N_DEV
        right = (my + 1) % N_DEV

        barrier_sem = pltpu.get_barrier_semaphore()
        pl.semaphore_signal(
            barrier_sem, inc=1, device_id=(left,),
            device_id_type=pl.DeviceIdType.MESH,
        )
        pl.semaphore_signal(
            barrier_sem, inc=1, device_id=(right,),
            device_id_type=pl.DeviceIdType.MESH,
        )
        pl.semaphore_wait(barrier_sem, 2)

        cp = pltpu.make_async_copy(
            x_ref, xg_ref.at[pl.ds(my * m_per, m_per), :], copy_sem
        )
        cp.start()
        cp.wait()

        for h in range(N_DEV - 1):
            origin = (my - h) % N_DEV
            rows = pl.ds(origin * m_per, m_per)
            rdma = pltpu.make_async_remote_copy(
                src_ref=xg_ref.at[rows, :],
                dst_ref=xg_ref.at[rows, :],
                send_sem=send_sems.at[h],
                recv_sem=recv_sems.at[h],
                device_id=(right,),
                device_id_type=pl.DeviceIdType.MESH,
            )
            rdma.start()
            rdma.wait()

    return pl.pallas_call(
        body,
        out_shape=jax.ShapeDtypeStruct((m_total, k), x.dtype),
        in_specs=[pl.BlockSpec(memory_space=pl.ANY)],
        out_specs=pl.BlockSpec(memory_space=pl.ANY),
        scratch_shapes=[
            pltpu.SemaphoreType.DMA,
            pltpu.SemaphoreType.DMA((N_DEV - 1,)),
            pltpu.SemaphoreType.DMA((N_DEV - 1,)),
        ],
        compiler_params=pltpu.CompilerParams(collective_id=0),
    )(x)


def kernel(x, w_mat):
    x_full = _ring_allgather(x)
    y = x_full @ w_mat
    return y * jax.nn.sigmoid(y)


# device time: 340419 ns/iter; 1.9652x vs baseline; 1.9652x over previous
import jax
import jax.numpy as jnp
from jax import lax
from jax.experimental import pallas as pl
from jax.experimental.pallas import tpu as pltpu

N_DEV = 4


def kernel(x, w_mat):
    m_per, k = x.shape
    n_per = w_mat.shape[1]
    m_total = N_DEV * m_per
    half = m_per // 2
    quar = m_per // 4

    def body(x_ref, w_ref, out_ref, xg, in_buf, out_buf,
             load_sems, store_sems, send_sems, rs1, rs2):
        my = lax.axis_index("i")
        left = (my - 1) % N_DEV
        right = (my + 1) % N_DEV
        diag = (my + 2) % N_DEV

        barrier_sem = pltpu.get_barrier_semaphore()
        for nbr in (left, right):
            pl.semaphore_signal(
                barrier_sem, inc=1, device_id=(nbr,),
                device_id_type=pl.DeviceIdType.MESH,
            )
        pl.semaphore_wait(barrier_sem, 2)

        sends = []

        def send(src_ref, src_row, dst_row, nrows, dev, si, rsem):
            rd = pltpu.make_async_remote_copy(
                src_ref=src_ref.at[pl.ds(src_row, nrows), :],
                dst_ref=xg.at[pl.ds(dst_row, nrows), :],
                send_sem=send_sems.at[si],
                recv_sem=rsem,
                device_id=(dev,),
                device_id_type=pl.DeviceIdType.MESH,
            )
            rd.start()
            sends.append(rd)

        def recv_wait(row_start, nrows, rsem):
            pltpu.make_async_remote_copy(
                src_ref=x_ref.at[pl.ds(0, nrows), :],
                dst_ref=xg.at[pl.ds(row_start, nrows), :],
                send_sem=send_sems.at[0],
                recv_sem=rsem,
                device_id=(left,),
                device_id_type=pl.DeviceIdType.MESH,
            ).wait_recv()

        si = 0
        for h in range(2):
            for dir_idx, dev in ((0, right), (1, left)):
                send(x_ref, h * half, my * m_per + h * half, half,
                     dev, si, rs1.at[dir_idx, h])
                si += 1

        state = {"pending": [None]}

        def compute_chunk(src_ref, src_row, out_row):
            ld = pltpu.make_async_copy(
                src_ref.at[pl.ds(src_row, m_per), :],
                in_buf, load_sems.at[0],
            )
            ld.start()
            ld.wait()
            if state["pending"][0] is not None:
                state["pending"][0].wait()
            y = jnp.dot(in_buf[...], w_ref[...],
                        preferred_element_type=jnp.float32)
            out_buf[...] = y * (1.0 / (1.0 + jnp.exp(-y)))
            st = pltpu.make_async_copy(
                out_buf,
                out_ref.at[pl.ds(out_row, m_per), :],
                store_sems.at[0],
            )
            st.start()
            state["pending"][0] = st

        compute_chunk(x_ref, 0, my * m_per)

        recv_wait(left * m_per, half, rs1.at[0, 0])
        for q in range(2):
            send(xg, left * m_per + q * quar, left * m_per + q * quar,
                 quar, right, 4 + q, rs2.at[0, q])

        recv_wait(right * m_per + half, half, rs1.at[1, 1])
        for q in range(2):
            send(xg, right * m_per + half + q * quar,
                 right * m_per + half + q * quar,
                 quar, left, 6 + q, rs2.at[1, q])

        recv_wait(left * m_per + half, half, rs1.at[0, 1])
        compute_chunk(xg, left * m_per, left * m_per)
        recv_wait(right * m_per, half, rs1.at[1, 0])
        compute_chunk(xg, right * m_per, right * m_per)

        for dir_idx, q in ((0, 0), (1, 2), (0, 1), (1, 3)):
            recv_wait(diag * m_per + q * quar, quar,
                      rs2.at[dir_idx, q % 2])
        compute_chunk(xg, diag * m_per, diag * m_per)

        for rd in sends:
            rd.wait_send()
        for p in state["pending"]:
            if p is not None:
                p.wait()

    out, _xg = pl.pallas_call(
        body,
        out_shape=[
            jax.ShapeDtypeStruct((m_total, n_per), jnp.float32),
            jax.ShapeDtypeStruct((m_total, k), jnp.float32),
        ],
        in_specs=[
            pl.BlockSpec(memory_space=pl.ANY),
            pl.BlockSpec(memory_space=pltpu.VMEM),
        ],
        out_specs=[
            pl.BlockSpec(memory_space=pl.ANY),
            pl.BlockSpec(memory_space=pl.ANY),
        ],
        scratch_shapes=[
            pltpu.VMEM((m_per, k), jnp.float32),
            pltpu.VMEM((m_per, n_per), jnp.float32),
            pltpu.SemaphoreType.DMA((1,)),
            pltpu.SemaphoreType.DMA((1,)),
            pltpu.SemaphoreType.DMA((8,)),
            pltpu.SemaphoreType.DMA((2, 2)),
            pltpu.SemaphoreType.DMA((2, 2)),
        ],
        compiler_params=pltpu.CompilerParams(
            collective_id=0,
            vmem_limit_bytes=60 * 1024 * 1024,
        ),
    )(x, w_mat)
    return out
